# baseline (speedup 1.0000x reference)
import functools

import jax
import jax.numpy as jnp
from jax import lax
from jax.experimental import pallas as pl
from jax.experimental.pallas import tpu as pltpu

_B = 2
_N = 40000
_DIM = 768
_G = 256
_M = 64
_NPAD = 40960
_ROWS = _NPAD // 128


def _fps_body(x_ref, cent_ref, dist_ref):
    flat = (lax.broadcasted_iota(jnp.int32, (_ROWS, 128), 0) * 128
            + lax.broadcasted_iota(jnp.int32, (_ROWS, 128), 1))
    valid = flat < _N
    lane_iota = lax.broadcasted_iota(jnp.int32, (1, 128), 1)
    for b in range(_B):
        dist_ref[b] = jnp.where(valid, jnp.float32(1e10), jnp.float32(-1.0))

    def step(i, fars):
        new_fars = []
        for b in range(_B):
            far = fars[b]
            row = far // 128
            lane = far - row * 128
            xr = x_ref[b, 0, pl.ds(row, 1), :]
            yr = x_ref[b, 1, pl.ds(row, 1), :]
            zr = x_ref[b, 2, pl.ds(row, 1), :]
            sel = lane_iota == lane
            cx = jnp.sum(jnp.where(sel, xr, 0.0))
            cy = jnp.sum(jnp.where(sel, yr, 0.0))
            cz = jnp.sum(jnp.where(sel, zr, 0.0))
            c2 = cx * cx + cy * cy + cz * cz
            rowvec = jnp.where(
                lane_iota == 0, cx,
                jnp.where(lane_iota == 1, cy,
                          jnp.where(lane_iota == 2, cz,
                                    jnp.where(lane_iota == 3, c2, 0.0))))
            cent_ref[pl.ds(b * _G + i, 1), :] = rowvec
            dx = x_ref[b, 0] - cx
            dy = x_ref[b, 1] - cy
            dz = x_ref[b, 2] - cz
            d = (dx * dx + dy * dy) + dz * dz
            nd = jnp.minimum(dist_ref[b], d)
            dist_ref[b] = nd
            maxv = jnp.max(jnp.max(nd, axis=0, keepdims=True))
            newfar = jnp.min(jnp.where(nd == maxv, flat, jnp.int32(2**30)))
            new_fars.append(newfar)
        return tuple(new_fars)

    lax.fori_loop(0, _G, step, tuple(jnp.int32(0) for _ in range(_B)))


def _run_fps(xpad):
    return pl.pallas_call(
        _fps_body,
        out_shape=jax.ShapeDtypeStruct((_B * _G, 128), jnp.float32),
        scratch_shapes=[pltpu.VMEM((_B, _ROWS, 128), jnp.float32)],
    )(xpad)


_FMAX = 3.4028235e38
_RT = 64
_NT = 5120


def _d2_body(cent_ref, xt_ref, x2_ref, d2_ref, cm_ref):
    lane8 = lax.broadcasted_iota(jnp.int32, (_RT, 8), 1)
    lhs = jnp.where(lane8 < 3, cent_ref[:, 0:8], 0.0)
    e = lax.dot_general(lhs, xt_ref[0], (((1,), (0,)), ((), ())),
                        precision=lax.Precision.DEFAULT,
                        preferred_element_type=jnp.float32)
    c2 = cent_ref[:, 3:4]
    d2 = (c2 - 2.0 * e) + x2_ref[0]
    col = lax.broadcasted_iota(jnp.int32, (_RT, _NT), 1) + pl.program_id(1) * _NT
    d2 = jnp.where(col < _N, d2, jnp.float32(_FMAX))
    d2_ref[...] = d2
    cm_ref[0] = jnp.min(d2.reshape(_RT, _NT // 128, 128), axis=2)


def _run_d2(cent2, xtp, x2p):
    gr = _B * _G // _RT
    gn = _NPAD // _NT
    return pl.pallas_call(
        _d2_body,
        grid=(gr, gn),
        in_specs=[
            pl.BlockSpec((_RT, 128), lambda r, n: (r, 0)),
            pl.BlockSpec((1, 8, _NT), lambda r, n: (r // (gr // _B), 0, n)),
            pl.BlockSpec((1, 1, _NT), lambda r, n: (r // (gr // _B), 0, n)),
        ],
        out_specs=[
            pl.BlockSpec((_RT, _NT), lambda r, n: (r, n)),
            pl.BlockSpec((1, _RT, _NT // 128), lambda r, n: (n, r, 0)),
        ],
        out_shape=[
            jax.ShapeDtypeStruct((_B * _G, _NPAD), jnp.float32),
            jax.ShapeDtypeStruct((gn, _B * _G, _NT // 128), jnp.float32),
        ],
    )(cent2, xtp, x2p)


def kernel(xyzs, pointcloud_features, level):
    Bb = _B
    xyz = xyzs[:, :_N, :]
    feats = pointcloud_features[:, :_N, :]

    xt = jnp.transpose(xyz, (0, 2, 1))
    xpad = jnp.pad(xt, ((0, 0), (0, 0), (0, _NPAD - _N)))
    xpad = xpad.reshape(_B, 3, _ROWS, 128)
    cent = _run_fps(xpad)
    center = cent[:, :3].reshape(_B, _G, 3)

    c2 = jnp.sum(center ** 2, axis=-1).reshape(_B * _G)
    lane = jnp.arange(128)[None, :]
    cent2 = jnp.where(lane == 3, c2[:, None], cent)
    xtp = jnp.pad(xt, ((0, 0), (0, 5), (0, _NPAD - _N)))
    x2p = jnp.pad(jnp.sum(xyz ** 2, axis=-1),
                  ((0, 0), (0, _NPAD - _N))).reshape(_B, 1, _NPAD)
    d2p, _cm = _run_d2(cent2, xtp, x2p)
    d2 = d2p.reshape(_B, _G, _NPAD)
    _, idx = lax.top_k(-lax.stop_gradient(d2), _M)
    bidx2 = jnp.arange(Bb)[:, None, None]
    nxyz = xyz[bidx2, idx] - center[:, :, None, :]
    scene_fts = feats[bidx2, idx].mean(-2)

    all_fts_mask = jnp.ones((Bb, _G), dtype=pointcloud_features.dtype)
    return scene_fts, all_fts_mask, center, nxyz


# speedup vs baseline: 7.8180x; 7.8180x over previous
import functools

import jax
import jax.numpy as jnp
from jax import lax
from jax.experimental import pallas as pl
from jax.experimental.pallas import tpu as pltpu
from jax.experimental.pallas import tpu_sc as plsc

_B = 2
_N = 40000
_DIM = 768
_G = 256
_M = 64
_NPAD = 40960
_ROWS = _NPAD // 128


def _fps_body(x_ref, cent_ref, dist_ref):
    flat = (lax.broadcasted_iota(jnp.int32, (_ROWS, 128), 0) * 128
            + lax.broadcasted_iota(jnp.int32, (_ROWS, 128), 1))
    valid = flat < _N
    lane_iota = lax.broadcasted_iota(jnp.int32, (1, 128), 1)
    for b in range(_B):
        dist_ref[b] = jnp.where(valid, jnp.float32(1e10), jnp.float32(-1.0))

    def step(i, fars):
        new_fars = []
        for b in range(_B):
            far = fars[b]
            row = far // 128
            lane = far - row * 128
            xr = x_ref[b, 0, pl.ds(row, 1), :]
            yr = x_ref[b, 1, pl.ds(row, 1), :]
            zr = x_ref[b, 2, pl.ds(row, 1), :]
            sel = lane_iota == lane
            cx = jnp.sum(jnp.where(sel, xr, 0.0))
            cy = jnp.sum(jnp.where(sel, yr, 0.0))
            cz = jnp.sum(jnp.where(sel, zr, 0.0))
            c2 = cx * cx + cy * cy + cz * cz
            rowvec = jnp.where(
                lane_iota == 0, cx,
                jnp.where(lane_iota == 1, cy,
                          jnp.where(lane_iota == 2, cz,
                                    jnp.where(lane_iota == 3, c2, 0.0))))
            cent_ref[pl.ds(b * _G + i, 1), :] = rowvec
            dx = x_ref[b, 0] - cx
            dy = x_ref[b, 1] - cy
            dz = x_ref[b, 2] - cz
            d = (dx * dx + dy * dy) + dz * dz
            nd = jnp.minimum(dist_ref[b], d)
            dist_ref[b] = nd
            maxv = jnp.max(jnp.max(nd, axis=0, keepdims=True))
            newfar = jnp.min(jnp.where(nd == maxv, flat, jnp.int32(2**30)))
            new_fars.append(newfar)
        return tuple(new_fars)

    lax.fori_loop(0, _G, step, tuple(jnp.int32(0) for _ in range(_B)))


def _run_fps(xpad):
    return pl.pallas_call(
        _fps_body,
        out_shape=jax.ShapeDtypeStruct((_B * _G, 128), jnp.float32),
        scratch_shapes=[pltpu.VMEM((_B, _ROWS, 128), jnp.float32)],
    )(xpad)


_FMAX = 3.4028235e38
_RT = 64
_NT = 5120


def _d2_body(cent_ref, xt_ref, x2_ref, d2_ref, cm_ref):
    lane8 = lax.broadcasted_iota(jnp.int32, (_RT, 8), 1)
    lhs = jnp.where(lane8 < 3, cent_ref[:, 0:8], 0.0)
    e = lax.dot_general(lhs, xt_ref[0], (((1,), (0,)), ((), ())),
                        precision=lax.Precision.DEFAULT,
                        preferred_element_type=jnp.float32)
    c2 = cent_ref[:, 3:4]
    d2 = (c2 - 2.0 * e) + x2_ref[0]
    col = lax.broadcasted_iota(jnp.int32, (_RT, _NT), 1) + pl.program_id(1) * _NT
    d2 = jnp.where(col < _N, d2, jnp.float32(_FMAX))
    d2_ref[...] = d2
    cm_ref[0] = jnp.min(d2.reshape(_RT, _NT // 128, 128), axis=2)


def _run_d2(cent2, xtp, x2p):
    gr = _B * _G // _RT
    gn = _NPAD // _NT
    return pl.pallas_call(
        _d2_body,
        grid=(gr, gn),
        in_specs=[
            pl.BlockSpec((_RT, 128), lambda r, n: (r, 0)),
            pl.BlockSpec((1, 8, _NT), lambda r, n: (r // (gr // _B), 0, n)),
            pl.BlockSpec((1, 1, _NT), lambda r, n: (r // (gr // _B), 0, n)),
        ],
        out_specs=[
            pl.BlockSpec((_RT, _NT), lambda r, n: (r, n)),
            pl.BlockSpec((1, _RT, _NT // 128), lambda r, n: (n, r, 0)),
        ],
        out_shape=[
            jax.ShapeDtypeStruct((_B * _G, _NPAD), jnp.float32),
            jax.ShapeDtypeStruct((gn, _B * _G, _NT // 128), jnp.float32),
        ],
    )(cent2, xtp, x2p)


_CAP = 2048
_NROW = _B * _G
_IMAX = 2147483647


def _key_i32(v):
    s = plsc.bitcast(v, jnp.int32)
    sh = lax.shift_right_arithmetic(s, 31)
    return s ^ (sh & _IMAX)


def _unkey_i32(k):
    sh = lax.shift_right_arithmetic(k, 31)
    return k ^ (sh & _IMAX)


def _bcast_last(x):
    return x.at[jnp.full((16,), 15, jnp.int32)].get(mode="promise_in_bounds")


def _vmax_all_i32(x):
    return _bcast_last(plsc.cummax(x))


def _vmin_all_i32(x):
    return ~_bcast_last(plsc.cummax(~x))


def _sc_body(d2_hbm, cm_hbm, cent_hbm, feats_hbm, xyzp_hbm,
             fts_hbm, nxyz_hbm,
             drow, cmv, candi, candk, seli, frows, acc, xrows, xr16, cvec,
             sem1, sem2):
    nc = 2
    wid = lax.axis_index("s") * nc + lax.axis_index("c")
    iota = jnp.arange(16, dtype=jnp.int32)

    def do_row(i, _):
        r = wid * 16 + i
        b = r // _G
        pltpu.sync_copy(d2_hbm.at[r], drow)
        pltpu.sync_copy(cm_hbm.at[r], cmv)
        pltpu.sync_copy(cent_hbm.at[r], cvec)

        gms = []
        for k in range(4):
            g = cmv[pl.ds(k * 16, 16)]
            for j in range(1, 5):
                g = jnp.minimum(g, cmv[pl.ds((k + 4 * j) * 16, 16)])
            gms.append(g)
        gmax = jnp.maximum(jnp.maximum(gms[0], gms[1]),
                           jnp.maximum(gms[2], gms[3]))
        ktv = _vmax_all_i32(_key_i32(gmax))
        tv = plsc.bitcast(_unkey_i32(ktv), jnp.float32)

        def fbody(j, offv):
            v = drow[pl.ds(j * 16, 16)]
            m = v <= tv
            mi = m.astype(jnp.int32)
            cs = plsc.cumsum(mi)
            pos = jnp.minimum(offv + cs - 1, _CAP - 1)
            plsc.store_scatter(candi, [pos], iota + j * 16, mask=m)
            return offv + plsc.all_reduce_population_count(m)

        offv = lax.fori_loop(0, _NPAD // 16, fbody,
                             jnp.zeros((16,), jnp.int32), unroll=8)
        cntv = jnp.minimum(offv, _CAP)
        count = cntv[0]
        nv = (count + 15) // 16

        def kbody(j, _):
            lanepos = iota + j * 16
            ok = lanepos < cntv
            idxv = candi[pl.ds(j * 16, 16)]
            vv = plsc.load_gather(drow, [idxv], mask=ok)
            kk = jnp.where(ok, _key_i32(vv), _IMAX)
            candk[pl.ds(j * 16, 16)] = kk
            return 0

        lax.fori_loop(0, nv, kbody, 0)

        def sbody(s, _):
            def p1(j, carry):
                mv, bj = carry
                v = candk[pl.ds(j * 16, 16)]
                upd = v < mv
                return (jnp.where(upd, v, mv),
                        jnp.where(upd, jnp.full((16,), j, jnp.int32), bj))

            mv, bj = lax.fori_loop(
                0, nv, p1,
                (jnp.full((16,), _IMAX, jnp.int32),
                 jnp.zeros((16,), jnp.int32)))
            mkv = _vmin_all_i32(mv)
            posv = bj * 16 + iota
            pm = jnp.where(mv == mkv, posv, _IMAX)
            p = _vmin_all_i32(pm)[0]
            jv = p // 16
            lane = p - jv * 16
            kvv = candk[pl.ds(jv * 16, 16)]
            candk[pl.ds(jv * 16, 16)] = jnp.where(iota == lane, _IMAX, kvv)
            iv = candi[pl.ds(jv * 16, 16)]
            oi = plsc.cumsum(jnp.where(iota == lane, iv, 0))[15]
            jv2 = s // 16
            old = seli[pl.ds(jv2 * 16, 16)]
            seli[pl.ds(jv2 * 16, 16)] = jnp.where(
                iota == (s - jv2 * 16), oi + b * _N, old)
            return 0

        lax.fori_loop(0, _M, sbody, 0)

        pltpu.async_copy(feats_hbm.at[seli], frows, sem1).wait()
        for dblk in range(_DIM // 16):
            acc[pl.ds(dblk * 16, 16)] = jnp.zeros((16,), jnp.float32)

        def abody(m, _):
            for dblk in range(_DIM // 16):
                sl = pl.ds(dblk * 16, 16)
                acc[sl] = acc[sl] + frows[m, sl]
            return 0

        lax.fori_loop(0, _M, abody, 0)
        for dblk in range(_DIM // 16):
            sl = pl.ds(dblk * 16, 16)
            acc[sl] = acc[sl] * jnp.float32(1.0 / _M)
        pltpu.sync_copy(acc, fts_hbm.at[r])

        pltpu.async_copy(xyzp_hbm.at[seli], xrows, sem2).wait()
        cmask = jnp.where(iota < 3, cvec[...], jnp.float32(0.0))

        def xbody(m, _):
            xr16[m, pl.ds(0, 16)] = xrows[m, pl.ds(0, 16)] - cmask
            return 0

        lax.fori_loop(0, _M, xbody, 0)
        pltpu.sync_copy(xr16, nxyz_hbm.at[r])
        return 0

    lax.fori_loop(0, 16, do_row, 0)


def _run_sc(d2p, cm, cent16, featsf, xyzp):
    mesh = plsc.VectorSubcoreMesh(core_axis_name="c", subcore_axis_name="s")
    f = pl.kernel(
        _sc_body,
        out_type=[
            jax.ShapeDtypeStruct((_NROW, _DIM), jnp.float32),
            jax.ShapeDtypeStruct((_NROW, _M, 16), jnp.float32),
        ],
        mesh=mesh,
        compiler_params=pltpu.CompilerParams(needs_layout_passes=False),
        scratch_types=[
            pltpu.VMEM((_NPAD,), jnp.float32),
            pltpu.VMEM((_ROWS,), jnp.float32),
            pltpu.VMEM((_CAP,), jnp.int32),
            pltpu.VMEM((_CAP,), jnp.int32),
            pltpu.VMEM((_M,), jnp.int32),
            pltpu.VMEM((_M, _DIM), jnp.float32),
            pltpu.VMEM((_DIM,), jnp.float32),
            pltpu.VMEM((_M, 128), jnp.float32),
            pltpu.VMEM((_M, 16), jnp.float32),
            pltpu.VMEM((16,), jnp.float32),
            pltpu.SemaphoreType.DMA,
            pltpu.SemaphoreType.DMA,
        ],
    )
    return f(d2p, cm, cent16, featsf, xyzp)


def kernel(xyzs, pointcloud_features, level):
    Bb = _B
    xyz = xyzs[:, :_N, :]
    feats = pointcloud_features[:, :_N, :]

    xt = jnp.transpose(xyz, (0, 2, 1))
    xpad = jnp.pad(xt, ((0, 0), (0, 0), (0, _NPAD - _N)))
    xpad = xpad.reshape(_B, 3, _ROWS, 128)
    cent = _run_fps(xpad)
    center = cent[:, :3].reshape(_B, _G, 3)

    c2 = jnp.sum(center ** 2, axis=-1).reshape(_B * _G)
    lane = jnp.arange(128)[None, :]
    cent2 = jnp.where(lane == 3, c2[:, None], cent)
    xtp = jnp.pad(xt, ((0, 0), (0, 5), (0, _NPAD - _N)))
    x2p = jnp.pad(jnp.sum(xyz ** 2, axis=-1),
                  ((0, 0), (0, _NPAD - _N))).reshape(_B, 1, _NPAD)
    d2p, cm3 = _run_d2(cent2, xtp, x2p)
    cm = jnp.transpose(cm3, (1, 0, 2)).reshape(_NROW, _ROWS)

    cent16 = cent[:, :16]
    featsf = feats.reshape(_B * _N, _DIM)
    xyzp = jnp.pad(xyz.reshape(_B * _N, 3), ((0, 0), (0, 125)))
    fts, nx16 = _run_sc(d2p, cm, cent16, featsf, xyzp)

    scene_fts = fts.reshape(_B, _G, _DIM)
    nxyz = nx16[:, :, :3].reshape(_B, _G, _M, 3)
    all_fts_mask = jnp.ones((Bb, _G), dtype=pointcloud_features.dtype)
    return scene_fts, all_fts_mask, center, nxyz


# speedup vs baseline: 8.1812x; 1.0465x over previous
import functools

import jax
import jax.numpy as jnp
from jax import lax
from jax.experimental import pallas as pl
from jax.experimental.pallas import tpu as pltpu
from jax.experimental.pallas import tpu_sc as plsc

_B = 2
_N = 40000
_DIM = 768
_G = 256
_M = 64
_NPAD = 40960
_ROWS = _NPAD // 128


def _fps_body(x_ref, cent_ref, dist_ref):
    flat = (lax.broadcasted_iota(jnp.int32, (_ROWS, 128), 0) * 128
            + lax.broadcasted_iota(jnp.int32, (_ROWS, 128), 1))
    valid = flat < _N
    lane_iota = lax.broadcasted_iota(jnp.int32, (1, 128), 1)
    for b in range(_B):
        dist_ref[b] = jnp.where(valid, jnp.float32(1e10), jnp.float32(-1.0))

    def step(i, fars):
        new_fars = []
        for b in range(_B):
            far = fars[b]
            row = far // 128
            lane = far - row * 128
            xr = x_ref[b, 0, pl.ds(row, 1), :]
            yr = x_ref[b, 1, pl.ds(row, 1), :]
            zr = x_ref[b, 2, pl.ds(row, 1), :]
            sel = lane_iota == lane
            cx = jnp.sum(jnp.where(sel, xr, 0.0))
            cy = jnp.sum(jnp.where(sel, yr, 0.0))
            cz = jnp.sum(jnp.where(sel, zr, 0.0))
            c2 = cx * cx + cy * cy + cz * cz
            rowvec = jnp.where(
                lane_iota == 0, cx,
                jnp.where(lane_iota == 1, cy,
                          jnp.where(lane_iota == 2, cz,
                                    jnp.where(lane_iota == 3, c2, 0.0))))
            cent_ref[pl.ds(b * _G + i, 1), :] = rowvec
            dx = x_ref[b, 0] - cx
            dy = x_ref[b, 1] - cy
            dz = x_ref[b, 2] - cz
            d = (dx * dx + dy * dy) + dz * dz
            nd = jnp.minimum(dist_ref[b], d)
            dist_ref[b] = nd
            maxv = jnp.max(jnp.max(nd, axis=0, keepdims=True))
            newfar = jnp.min(jnp.where(nd == maxv, flat, jnp.int32(2**30)))
            new_fars.append(newfar)
        return tuple(new_fars)

    lax.fori_loop(0, _G, step, tuple(jnp.int32(0) for _ in range(_B)))


def _run_fps(xpad):
    return pl.pallas_call(
        _fps_body,
        out_shape=jax.ShapeDtypeStruct((_B * _G, 128), jnp.float32),
        scratch_shapes=[pltpu.VMEM((_B, _ROWS, 128), jnp.float32)],
    )(xpad)


_FMAX = 3.4028235e38
_RT = 64
_NT = 5120


def _d2_body(cent_ref, xt_ref, x2_ref, d2_ref, cm_ref):
    lane8 = lax.broadcasted_iota(jnp.int32, (_RT, 8), 1)
    lhs = jnp.where(lane8 < 3, cent_ref[:, 0:8], 0.0)
    e = lax.dot_general(lhs, xt_ref[0], (((1,), (0,)), ((), ())),
                        precision=lax.Precision.DEFAULT,
                        preferred_element_type=jnp.float32)
    c2 = cent_ref[:, 3:4]
    d2 = (c2 - 2.0 * e) + x2_ref[0]
    col = lax.broadcasted_iota(jnp.int32, (_RT, _NT), 1) + pl.program_id(1) * _NT
    d2 = jnp.where(col < _N, d2, jnp.float32(_FMAX))
    d2r = d2.reshape(_RT, _NT // 128, 128)
    d2_ref[...] = d2r
    cm_ref[0] = jnp.min(d2r, axis=2)


def _run_d2(cent2, xtp, x2p):
    gr = _B * _G // _RT
    gn = _NPAD // _NT
    return pl.pallas_call(
        _d2_body,
        grid=(gr, gn),
        in_specs=[
            pl.BlockSpec((_RT, 128), lambda r, n: (r, 0)),
            pl.BlockSpec((1, 8, _NT), lambda r, n: (r // (gr // _B), 0, n)),
            pl.BlockSpec((1, 1, _NT), lambda r, n: (r // (gr // _B), 0, n)),
        ],
        out_specs=[
            pl.BlockSpec((_RT, _NT // 128, 128), lambda r, n: (r, n, 0)),
            pl.BlockSpec((1, _RT, _NT // 128), lambda r, n: (n, r, 0)),
        ],
        out_shape=[
            jax.ShapeDtypeStruct((_B * _G, _ROWS, 128), jnp.float32),
            jax.ShapeDtypeStruct((gn, _B * _G, _NT // 128), jnp.float32),
        ],
    )(cent2, xtp, x2p)


_CAP = 2048
_NROW = _B * _G
_IMAX = 2147483647


def _key_i32(v):
    s = plsc.bitcast(v, jnp.int32)
    sh = lax.shift_right_arithmetic(s, 31)
    return s ^ (sh & _IMAX)


def _unkey_i32(k):
    sh = lax.shift_right_arithmetic(k, 31)
    return k ^ (sh & _IMAX)


def _bcast_last(x):
    return x.at[jnp.full((16,), 15, jnp.int32)].get(mode="promise_in_bounds")


def _vmax_all_i32(x):
    return _bcast_last(plsc.cummax(x))


def _vmin_all_i32(x):
    return ~_bcast_last(plsc.cummax(~x))


def _sc_body(d2_hbm, cm_hbm, cent_hbm, feats_hbm, xyzp_hbm,
             fts_hbm, nxyz_hbm,
             drow, cmv, candi, candk, seli, frows, acc, xrows, xr16, cvec,
             sem1, sem2):
    nc = 2
    wid = lax.axis_index("s") * nc + lax.axis_index("c")
    iota = jnp.arange(16, dtype=jnp.int32)

    def do_row(i, _):
        r = wid * 16 + i
        b = r // _G
        pltpu.sync_copy(d2_hbm.at[r], drow)
        pltpu.sync_copy(cm_hbm.at[r], cmv)
        pltpu.sync_copy(cent_hbm.at[r], cvec)

        gms = []
        for k in range(4):
            g = cmv[pl.ds(k * 16, 16)]
            for j in range(1, 5):
                g = jnp.minimum(g, cmv[pl.ds((k + 4 * j) * 16, 16)])
            gms.append(g)
        gmax = jnp.maximum(jnp.maximum(gms[0], gms[1]),
                           jnp.maximum(gms[2], gms[3]))
        ktv = _vmax_all_i32(_key_i32(gmax))
        tv = plsc.bitcast(_unkey_i32(ktv), jnp.float32)

        def fbody(c, offv):
            base = c * 128
            for k in range(8):
                v = drow[c, pl.ds(k * 16, 16)]
                m = v <= tv
                cs = plsc.cumsum(m.astype(jnp.int32))
                pos = jnp.minimum(offv + cs - 1, _CAP - 1)
                plsc.store_scatter(candi, [pos], iota + (base + k * 16),
                                   mask=m)
                offv = offv + plsc.all_reduce_population_count(m)
            return offv

        offv = lax.fori_loop(0, _ROWS, fbody, jnp.zeros((16,), jnp.int32))
        cntv = jnp.minimum(offv, _CAP)
        count = cntv[0]
        nv = (count + 15) // 16

        def kbody(j, _):
            lanepos = iota + j * 16
            ok = lanepos < cntv
            idxv = candi[pl.ds(j * 16, 16)]
            vv = plsc.load_gather(drow, [idxv >> 7, idxv & 127], mask=ok)
            kk = jnp.where(ok, _key_i32(vv), _IMAX)
            candk[pl.ds(j * 16, 16)] = kk
            return 0

        lax.fori_loop(0, nv, kbody, 0)

        def sbody(s, _):
            def p1(j, carry):
                mv, bj = carry
                v = candk[pl.ds(j * 16, 16)]
                upd = v < mv
                return (jnp.where(upd, v, mv),
                        jnp.where(upd, jnp.full((16,), j, jnp.int32), bj))

            mv, bj = lax.fori_loop(
                0, nv, p1,
                (jnp.full((16,), _IMAX, jnp.int32),
                 jnp.zeros((16,), jnp.int32)))
            mkv = _vmin_all_i32(mv)
            posv = bj * 16 + iota
            pm = jnp.where(mv == mkv, posv, _IMAX)
            p = _vmin_all_i32(pm)[0]
            jv = p // 16
            lane = p - jv * 16
            kvv = candk[pl.ds(jv * 16, 16)]
            candk[pl.ds(jv * 16, 16)] = jnp.where(iota == lane, _IMAX, kvv)
            iv = candi[pl.ds(jv * 16, 16)]
            oi = plsc.cumsum(jnp.where(iota == lane, iv, 0))[15]
            jv2 = s // 16
            old = seli[pl.ds(jv2 * 16, 16)]
            seli[pl.ds(jv2 * 16, 16)] = jnp.where(
                iota == (s - jv2 * 16), oi + b * _N, old)
            return 0

        lax.fori_loop(0, _M, sbody, 0)

        cp1 = pltpu.async_copy(feats_hbm.at[seli], frows, sem1)
        cp2 = pltpu.async_copy(xyzp_hbm.at[seli], xrows, sem2)
        for dblk in range(_DIM // 16):
            acc[pl.ds(dblk * 16, 16)] = jnp.zeros((16,), jnp.float32)
        cp1.wait()

        def abody(m, _):
            for dblk in range(_DIM // 16):
                sl = pl.ds(dblk * 16, 16)
                plsc.addupdate(acc.at[sl], frows[m, sl])
            return 0

        lax.fori_loop(0, _M, abody, 0, unroll=2)
        for dblk in range(_DIM // 16):
            sl = pl.ds(dblk * 16, 16)
            acc[sl] = acc[sl] * jnp.float32(1.0 / _M)
        pltpu.sync_copy(acc, fts_hbm.at[r])

        cp2.wait()
        cmask = jnp.where(iota < 3, cvec[...], jnp.float32(0.0))

        def xbody(m, _):
            xr16[m, pl.ds(0, 16)] = xrows[m, pl.ds(0, 16)] - cmask
            return 0

        lax.fori_loop(0, _M, xbody, 0)
        pltpu.sync_copy(xr16, nxyz_hbm.at[r])
        return 0

    lax.fori_loop(0, 16, do_row, 0)


def _run_sc(d2p, cm, cent16, featsf, xyzp):
    mesh = plsc.VectorSubcoreMesh(core_axis_name="c", subcore_axis_name="s")
    f = pl.kernel(
        _sc_body,
        out_type=[
            jax.ShapeDtypeStruct((_NROW, _DIM), jnp.float32),
            jax.ShapeDtypeStruct((_NROW, _M, 16), jnp.float32),
        ],
        mesh=mesh,
        compiler_params=pltpu.CompilerParams(needs_layout_passes=False),
        scratch_types=[
            pltpu.VMEM((_ROWS, 128), jnp.float32),
            pltpu.VMEM((_ROWS,), jnp.float32),
            pltpu.VMEM((_CAP,), jnp.int32),
            pltpu.VMEM((_CAP,), jnp.int32),
            pltpu.VMEM((_M,), jnp.int32),
            pltpu.VMEM((_M, _DIM), jnp.float32),
            pltpu.VMEM((_DIM,), jnp.float32),
            pltpu.VMEM((_M, 128), jnp.float32),
            pltpu.VMEM((_M, 16), jnp.float32),
            pltpu.VMEM((16,), jnp.float32),
            pltpu.SemaphoreType.DMA,
            pltpu.SemaphoreType.DMA,
        ],
    )
    return f(d2p, cm, cent16, featsf, xyzp)


def kernel(xyzs, pointcloud_features, level):
    Bb = _B
    xyz = xyzs[:, :_N, :]
    feats = pointcloud_features[:, :_N, :]

    xt = jnp.transpose(xyz, (0, 2, 1))
    xpad = jnp.pad(xt, ((0, 0), (0, 0), (0, _NPAD - _N)))
    xpad = xpad.reshape(_B, 3, _ROWS, 128)
    cent = _run_fps(xpad)
    center = cent[:, :3].reshape(_B, _G, 3)

    c2 = jnp.sum(center ** 2, axis=-1).reshape(_B * _G)
    lane = jnp.arange(128)[None, :]
    cent2 = jnp.where(lane == 3, c2[:, None], cent)
    xtp = jnp.pad(xt, ((0, 0), (0, 5), (0, _NPAD - _N)))
    x2p = jnp.pad(jnp.sum(xyz ** 2, axis=-1),
                  ((0, 0), (0, _NPAD - _N))).reshape(_B, 1, _NPAD)
    d2p, cm3 = _run_d2(cent2, xtp, x2p)
    cm = jnp.transpose(cm3, (1, 0, 2)).reshape(_NROW, _ROWS)

    cent16 = cent[:, :16]
    featsf = feats.reshape(_B * _N, _DIM)
    xyzp = jnp.pad(xyz.reshape(_B * _N, 3), ((0, 0), (0, 125)))
    fts, nx16 = _run_sc(d2p, cm, cent16, featsf, xyzp)

    scene_fts = fts.reshape(_B, _G, _DIM)
    nxyz = nx16[:, :, :3].reshape(_B, _G, _M, 3)
    all_fts_mask = jnp.ones((Bb, _G), dtype=pointcloud_features.dtype)
    return scene_fts, all_fts_mask, center, nxyz


# speedup vs baseline: 8.6591x; 1.0584x over previous
import functools

import jax
import jax.numpy as jnp
from jax import lax
from jax.experimental import pallas as pl
from jax.experimental.pallas import tpu as pltpu
from jax.experimental.pallas import tpu_sc as plsc

_B = 2
_N = 40000
_DIM = 768
_G = 256
_M = 64
_NPAD = 40960
_ROWS = _NPAD // 128


def _fps_body(x_ref, cent_ref, dist_ref):
    flat = (lax.broadcasted_iota(jnp.int32, (_ROWS, 128), 0) * 128
            + lax.broadcasted_iota(jnp.int32, (_ROWS, 128), 1))
    valid = flat < _N
    lane_iota = lax.broadcasted_iota(jnp.int32, (1, 128), 1)
    for b in range(_B):
        dist_ref[b] = jnp.where(valid, jnp.float32(1e10), jnp.float32(-1.0))

    def step(i, fars):
        new_fars = []
        for b in range(_B):
            far = fars[b]
            row = far // 128
            lane = far - row * 128
            xr = x_ref[b, 0, pl.ds(row, 1), :]
            yr = x_ref[b, 1, pl.ds(row, 1), :]
            zr = x_ref[b, 2, pl.ds(row, 1), :]
            sel = lane_iota == lane
            cx = jnp.sum(jnp.where(sel, xr, 0.0))
            cy = jnp.sum(jnp.where(sel, yr, 0.0))
            cz = jnp.sum(jnp.where(sel, zr, 0.0))
            c2 = cx * cx + cy * cy + cz * cz
            rowvec = jnp.where(
                lane_iota == 0, cx,
                jnp.where(lane_iota == 1, cy,
                          jnp.where(lane_iota == 2, cz,
                                    jnp.where(lane_iota == 3, c2, 0.0))))
            cent_ref[pl.ds(b * _G + i, 1), :] = rowvec
            dx = x_ref[b, 0] - cx
            dy = x_ref[b, 1] - cy
            dz = x_ref[b, 2] - cz
            d = (dx * dx + dy * dy) + dz * dz
            nd = jnp.minimum(dist_ref[b], d)
            dist_ref[b] = nd
            maxv = jnp.max(jnp.max(nd, axis=0, keepdims=True))
            newfar = jnp.min(jnp.where(nd == maxv, flat, jnp.int32(2**30)))
            new_fars.append(newfar)
        return tuple(new_fars)

    lax.fori_loop(0, _G, step, tuple(jnp.int32(0) for _ in range(_B)))


def _run_fps(xpad):
    return pl.pallas_call(
        _fps_body,
        out_shape=jax.ShapeDtypeStruct((_B * _G, 128), jnp.float32),
        scratch_shapes=[pltpu.VMEM((_B, _ROWS, 128), jnp.float32)],
    )(xpad)


_FMAX = 3.4028235e38
_RT = 64
_NT = 5120


def _d2_body(cent_ref, xt_ref, x2_ref, d2_ref, cm_ref):
    lane8 = lax.broadcasted_iota(jnp.int32, (_RT, 8), 1)
    lhs = jnp.where(lane8 < 3, cent_ref[:, 0:8], 0.0)
    e = lax.dot_general(lhs, xt_ref[0], (((1,), (0,)), ((), ())),
                        precision=lax.Precision.DEFAULT,
                        preferred_element_type=jnp.float32)
    c2 = cent_ref[:, 3:4]
    d2 = (c2 - 2.0 * e) + x2_ref[0]
    col = lax.broadcasted_iota(jnp.int32, (_RT, _NT), 1) + pl.program_id(1) * _NT
    d2 = jnp.where(col < _N, d2, jnp.float32(_FMAX))
    d2r = d2.reshape(_RT, _NT // 128, 128)
    d2_ref[...] = d2r
    cm_ref[0] = jnp.min(d2r, axis=2)


def _run_d2(cent2, xtp, x2p):
    gr = _B * _G // _RT
    gn = _NPAD // _NT
    return pl.pallas_call(
        _d2_body,
        grid=(gr, gn),
        in_specs=[
            pl.BlockSpec((_RT, 128), lambda r, n: (r, 0)),
            pl.BlockSpec((1, 8, _NT), lambda r, n: (r // (gr // _B), 0, n)),
            pl.BlockSpec((1, 1, _NT), lambda r, n: (r // (gr // _B), 0, n)),
        ],
        out_specs=[
            pl.BlockSpec((_RT, _NT // 128, 128), lambda r, n: (r, n, 0)),
            pl.BlockSpec((1, _RT, _NT // 128), lambda r, n: (n, r, 0)),
        ],
        out_shape=[
            jax.ShapeDtypeStruct((_B * _G, _ROWS, 128), jnp.float32),
            jax.ShapeDtypeStruct((gn, _B * _G, _NT // 128), jnp.float32),
        ],
    )(cent2, xtp, x2p)


_CAP = 2048
_NROW = _B * _G
_IMAX = 2147483647


def _key_i32(v):
    s = plsc.bitcast(v, jnp.int32)
    sh = lax.shift_right_arithmetic(s, 31)
    return s ^ (sh & _IMAX)


def _unkey_i32(k):
    sh = lax.shift_right_arithmetic(k, 31)
    return k ^ (sh & _IMAX)


def _bcast_last(x):
    return x.at[jnp.full((16,), 15, jnp.int32)].get(mode="promise_in_bounds")


def _vmax_all_i32(x):
    return _bcast_last(plsc.cummax(x))


def _vmin_all_i32(x):
    return ~_bcast_last(plsc.cummax(~x))


def _sc_body(d2_hbm, cm_hbm, cent_hbm, feats_hbm, xyzp_hbm,
             fts_hbm, nxyz_hbm,
             drow, cmv, candi, candk, seli, frows, acc, xrows, xr16, cvec,
             sem1, sem2):
    nc = 2
    wid = lax.axis_index("s") * nc + lax.axis_index("c")
    iota = jnp.arange(16, dtype=jnp.int32)

    def do_row(i, _):
        r = wid * 16 + i
        b = r // _G
        pltpu.sync_copy(d2_hbm.at[r], drow)
        pltpu.sync_copy(cm_hbm.at[r], cmv)
        pltpu.sync_copy(cent_hbm.at[r], cvec)

        gms = []
        for k in range(4):
            g = cmv[pl.ds(k * 16, 16)]
            for j in range(1, 5):
                g = jnp.minimum(g, cmv[pl.ds((k + 4 * j) * 16, 16)])
            gms.append(g)
        gmax = jnp.maximum(jnp.maximum(gms[0], gms[1]),
                           jnp.maximum(gms[2], gms[3]))
        ktv = _vmax_all_i32(_key_i32(gmax))
        tv = plsc.bitcast(_unkey_i32(ktv), jnp.float32)

        def fbody(c, offv):
            base = c * 128
            for k in range(8):
                v = drow[c, pl.ds(k * 16, 16)]
                m = v <= tv
                cs = plsc.cumsum(m.astype(jnp.int32))
                pos = jnp.minimum(offv + cs - 1, _CAP - 1)
                plsc.store_scatter(candi, [pos], iota + (base + k * 16),
                                   mask=m)
                offv = offv + plsc.all_reduce_population_count(m)
            return offv

        offv = lax.fori_loop(0, _ROWS, fbody, jnp.zeros((16,), jnp.int32))
        cntv = jnp.minimum(offv, _CAP)
        count = cntv[0]
        nv = (count + 15) // 16

        def kbody(j, _):
            lanepos = iota + j * 16
            ok = lanepos < cntv
            idxv = candi[pl.ds(j * 16, 16)]
            vv = plsc.load_gather(drow, [idxv >> 7, idxv & 127], mask=ok)
            kk = jnp.where(ok, _key_i32(vv), _IMAX)
            candk[pl.ds(j * 16, 16)] = kk
            return 0

        lax.fori_loop(0, nv, kbody, 0)
        for k in range(3):
            candk[pl.ds((nv + k) * 16, 16)] = jnp.full((16,), _IMAX,
                                                       jnp.int32)
        nvb = (nv + 3) // 4

        def sbody(s, _):
            def p1(jb, carry):
                mv, bj = carry
                for k in range(4):
                    j = jb * 4 + k
                    v = candk[pl.ds(j * 16, 16)]
                    upd = v < mv
                    mv = jnp.where(upd, v, mv)
                    bj = jnp.where(upd, jnp.full((16,), j, jnp.int32), bj)
                return mv, bj

            mv, bj = lax.fori_loop(
                0, nvb, p1,
                (jnp.full((16,), _IMAX, jnp.int32),
                 jnp.zeros((16,), jnp.int32)))
            mkv = _vmin_all_i32(mv)
            posv = bj * 16 + iota
            pm = jnp.where(mv == mkv, posv, _IMAX)
            p = _vmin_all_i32(pm)[0]
            jv = p // 16
            lane = p - jv * 16
            kvv = candk[pl.ds(jv * 16, 16)]
            candk[pl.ds(jv * 16, 16)] = jnp.where(iota == lane, _IMAX, kvv)
            iv = candi[pl.ds(jv * 16, 16)]
            oi = plsc.cumsum(jnp.where(iota == lane, iv, 0))[15]
            jv2 = s // 16
            old = seli[pl.ds(jv2 * 16, 16)]
            seli[pl.ds(jv2 * 16, 16)] = jnp.where(
                iota == (s - jv2 * 16), oi + b * _N, old)
            return 0

        lax.fori_loop(0, _M, sbody, 0)

        cp1 = pltpu.async_copy(feats_hbm.at[seli], frows, sem1)
        cp2 = pltpu.async_copy(xyzp_hbm.at[seli], xrows, sem2)
        for dblk in range(_DIM // 16):
            acc[pl.ds(dblk * 16, 16)] = jnp.zeros((16,), jnp.float32)
        cp1.wait()

        def abody(m, _):
            for dblk in range(_DIM // 16):
                sl = pl.ds(dblk * 16, 16)
                plsc.addupdate(acc.at[sl], frows[m, sl])
            return 0

        lax.fori_loop(0, _M, abody, 0, unroll=4)
        for dblk in range(_DIM // 16):
            sl = pl.ds(dblk * 16, 16)
            acc[sl] = acc[sl] * jnp.float32(1.0 / _M)
        pltpu.sync_copy(acc, fts_hbm.at[r])

        cp2.wait()
        cmask = jnp.where(iota < 3, cvec[...], jnp.float32(0.0))

        def xbody(m, _):
            xr16[m, pl.ds(0, 16)] = xrows[m, pl.ds(0, 16)] - cmask
            return 0

        lax.fori_loop(0, _M, xbody, 0)
        pltpu.sync_copy(xr16, nxyz_hbm.at[r])
        return 0

    lax.fori_loop(0, 16, do_row, 0)


def _run_sc(d2p, cm, cent16, featsf, xyzp):
    mesh = plsc.VectorSubcoreMesh(core_axis_name="c", subcore_axis_name="s")
    f = pl.kernel(
        _sc_body,
        out_type=[
            jax.ShapeDtypeStruct((_NROW, _DIM), jnp.float32),
            jax.ShapeDtypeStruct((_NROW, _M, 16), jnp.float32),
        ],
        mesh=mesh,
        compiler_params=pltpu.CompilerParams(needs_layout_passes=False),
        scratch_types=[
            pltpu.VMEM((_ROWS, 128), jnp.float32),
            pltpu.VMEM((_ROWS,), jnp.float32),
            pltpu.VMEM((_CAP,), jnp.int32),
            pltpu.VMEM((_CAP + 48,), jnp.int32),
            pltpu.VMEM((_M,), jnp.int32),
            pltpu.VMEM((_M, _DIM), jnp.float32),
            pltpu.VMEM((_DIM,), jnp.float32),
            pltpu.VMEM((_M, 128), jnp.float32),
            pltpu.VMEM((_M, 16), jnp.float32),
            pltpu.VMEM((16,), jnp.float32),
            pltpu.SemaphoreType.DMA,
            pltpu.SemaphoreType.DMA,
        ],
    )
    return f(d2p, cm, cent16, featsf, xyzp)


def kernel(xyzs, pointcloud_features, level):
    Bb = _B
    xyz = xyzs[:, :_N, :]
    feats = pointcloud_features[:, :_N, :]

    xt = jnp.transpose(xyz, (0, 2, 1))
    xpad = jnp.pad(xt, ((0, 0), (0, 0), (0, _NPAD - _N)))
    xpad = xpad.reshape(_B, 3, _ROWS, 128)
    cent = _run_fps(xpad)
    center = cent[:, :3].reshape(_B, _G, 3)

    c2 = jnp.sum(center ** 2, axis=-1).reshape(_B * _G)
    lane = jnp.arange(128)[None, :]
    cent2 = jnp.where(lane == 3, c2[:, None], cent)
    xtp = jnp.pad(xt, ((0, 0), (0, 5), (0, _NPAD - _N)))
    x2p = jnp.pad(jnp.sum(xyz ** 2, axis=-1),
                  ((0, 0), (0, _NPAD - _N))).reshape(_B, 1, _NPAD)
    d2p, cm3 = _run_d2(cent2, xtp, x2p)
    cm = jnp.transpose(cm3, (1, 0, 2)).reshape(_NROW, _ROWS)

    cent16 = cent[:, :16]
    featsf = feats.reshape(_B * _N, _DIM)
    xyzp = jnp.pad(xyz.reshape(_B * _N, 3), ((0, 0), (0, 125)))
    fts, nx16 = _run_sc(d2p, cm, cent16, featsf, xyzp)

    scene_fts = fts.reshape(_B, _G, _DIM)
    nxyz = nx16[:, :, :3].reshape(_B, _G, _M, 3)
    all_fts_mask = jnp.ones((Bb, _G), dtype=pointcloud_features.dtype)
    return scene_fts, all_fts_mask, center, nxyz
